# Initial kernel scaffold; baseline (speedup 1.0000x reference)
import jax
import jax.numpy as jnp
from jax.experimental import pallas as pl


def kernel(x, edge_index, edge_attr, params):
    raise NotImplementedError("write your pallas kernel here")


# speedup vs baseline: 5.3376x; 5.3376x over previous
import jax
import jax.numpy as jnp
from jax import lax
from jax.experimental import pallas as pl
from jax.experimental.pallas import tpu as pltpu
from jax.experimental.pallas import tpu_sc as plsc

N_NODES = 50000
NP = N_NODES + 8
CHUNK = 128
BCH = 8
BLOCK = CHUNK * BCH
NTILES = 32
CPT = 392
NBLK = CPT // BCH
EP = NTILES * CPT * CHUNK
BLKN = 2000

_f32 = jnp.float32
_i32 = jnp.int32


def _edge_sc(Cy, Ce, Oe, pool, write_ea):
  Cpre = Cy + Oe
  Cout = Cpre // 3 if pool else Cpre
  Pw = Oe * Ce + Oe + Cy
  Pp = ((Pw + 15) // 16) * 16
  mesh = plsc.VectorSubcoreMesh(core_axis_name="c", subcore_axis_name="s")
  out_type = []
  if write_ea:
    out_type.append(jax.ShapeDtypeStruct((EP, Cout), _f32))
  out_type.append(jax.ShapeDtypeStruct((2, NP, Cout), _f32))
  scratch = [
      pltpu.VMEM((BCH, CHUNK), _i32),
      pltpu.VMEM((BCH, CHUNK), _i32),
      pltpu.VMEM((BLOCK * Ce,), _f32),
      pltpu.VMEM((BLOCK, Cy), _f32),
      pltpu.VMEM((BLOCK, Cy), _f32),
      pltpu.VMEM((BLOCK, Cout), _f32),
      pltpu.VMEM((Pp,), _f32),
      pltpu.VMEM_SHARED((NP, Cout), _f32),
      pltpu.SemaphoreType.DMA,
  ]

  def body(y_h, i0_h, i1_h, ea_h, w_h, z_h, *rest):
    if write_ea:
      eaout_h, agg_h = rest[0], rest[1]
      scr = rest[2:]
    else:
      agg_h = rest[0]
      scr = rest[1:]
    idx0, idx1, eab, r0, r1, outb, wvm, aggs, sem = scr
    cid = lax.axis_index("c")
    sid = lax.axis_index("s")
    wid = sid * 2 + cid

    pltpu.sync_copy(w_h, wvm)
    ws = []
    for k in range(Pp // 16):
      v = wvm[pl.ds(k * 16, 16)]
      ws.extend(v[j] for j in range(16))

    @pl.when(sid == 0)
    def _zero():
      pltpu.sync_copy(z_h, aggs)

    plsc.subcore_barrier()

    iota16 = lax.iota(_i32, 16)
    tile_ch = wid * CPT

    def block_body(blk, carry):
      ch0 = tile_ch + blk * BCH
      e0 = ch0 * CHUNK
      pltpu.sync_copy(i0_h.at[pl.ds(ch0, BCH)], idx0)
      pltpu.sync_copy(i1_h.at[pl.ds(ch0, BCH)], idx1)
      pltpu.sync_copy(ea_h.at[pl.ds(e0 * Ce, BLOCK * Ce)], eab)
      descs = []
      for k in range(BCH):
        descs.append(pltpu.async_copy(
            y_h.at[idx0.at[k]], r0.at[pl.ds(k * CHUNK, CHUNK)], sem))
        descs.append(pltpu.async_copy(
            y_h.at[idx1.at[k]], r1.at[pl.ds(k * CHUNK, CHUNK)], sem))
      for d in descs:
        d.wait()

      def group(g, c2):
        rows = g * 16 + iota16
        erows = rows * Ce
        hs = [jnp.zeros((16,), _f32) + ws[Oe * Ce + j] for j in range(Oe)]
        for c in range(Ce):
          a = plsc.load_gather(eab, [erows + c])
          for j in range(Oe):
            hs[j] = hs[j] + a * ws[j * Ce + c]

        def uch(i):
          if i >= Cy:
            return hs[i - Cy]
          a0 = plsc.load_gather(r0, [rows, jnp.full((16,), i, _i32)])
          a1 = plsc.load_gather(r1, [rows, jnp.full((16,), i, _i32)])
          return a0 + a1 + ws[Oe * Ce + Oe + i]

        for k in range(Cout):
          if pool:
            v = jnp.maximum(jnp.maximum(uch(3 * k), uch(3 * k + 1)),
                            uch(3 * k + 2))
          else:
            v = uch(k)
          v = jnp.maximum(v, 0.0)
          plsc.store_scatter(outb, [rows, jnp.full((16,), k, _i32)], v)
        return c2

      lax.fori_loop(0, BLOCK // 16, group, 0)
      if write_ea:
        pltpu.sync_copy(outb, eaout_h.at[pl.ds(e0, BLOCK)])
      for k in range(BCH):
        pltpu.sync_copy(outb.at[pl.ds(k * CHUNK, CHUNK)],
                        aggs.at[idx0.at[k]], add=True)
      return carry

    lax.fori_loop(0, NBLK, block_body, 0)
    plsc.subcore_barrier()

    @pl.when(sid == 0)
    def _flush():
      pltpu.sync_copy(aggs, agg_h.at[cid])

  return pl.kernel(
      body, out_type=out_type, mesh=mesh, scratch_types=scratch,
      compiler_params=pltpu.CompilerParams(needs_layout_passes=False,
                                           use_tc_tiling_on_sc=False))


def _tc_linear(xx, wt):
  n, ci = xx.shape
  o = wt.shape[1]

  def body(x_ref, w_ref, o_ref):
    o_ref[...] = jnp.dot(x_ref[...], w_ref[...],
                         preferred_element_type=_f32)

  return pl.pallas_call(
      body,
      grid=(n // BLKN,),
      in_specs=[pl.BlockSpec((BLKN, ci), lambda i: (i, 0)),
                pl.BlockSpec((ci, o), lambda i: (0, 0))],
      out_specs=pl.BlockSpec((BLKN, o), lambda i: (i, 0)),
      out_shape=jax.ShapeDtypeStruct((n, o), _f32),
  )(xx, wt)


def _tc_node(xx, a0, a1, wxt, bx, wet, be, wyt):
  n, ci = xx.shape
  ca = a0.shape[1]
  ga = wxt.shape[1] // 3
  gb = wet.shape[1] // 3
  cn = ga + gb
  oy = wyt.shape[1]

  def body(x_ref, a0_ref, a1_ref, wx_ref, bx_ref, we_ref, be_ref, wy_ref,
           ox_ref, oy_ref):
    xv = x_ref[...]
    agg = a0_ref[...] + a1_ref[...]
    xa = jnp.dot(xv, wx_ref[...], preferred_element_type=_f32) + bx_ref[...]
    xe = jnp.dot(agg, we_ref[...], preferred_element_type=_f32) + be_ref[...]
    xa = jnp.maximum(xa, 0.0)
    xe = jnp.maximum(xe, 0.0)
    ra = jnp.maximum(jnp.maximum(xa[:, :ga], xa[:, ga:2 * ga]),
                     xa[:, 2 * ga:])
    rb = jnp.maximum(jnp.maximum(xe[:, :gb], xe[:, gb:2 * gb]),
                     xe[:, 2 * gb:])
    xn = jnp.concatenate([ra, rb], axis=1)
    ox_ref[...] = xn
    oy_ref[...] = jnp.dot(xn, wy_ref[...], preferred_element_type=_f32)

  return pl.pallas_call(
      body,
      grid=(n // BLKN,),
      in_specs=[
          pl.BlockSpec((BLKN, ci), lambda i: (i, 0)),
          pl.BlockSpec((BLKN, ca), lambda i: (i, 0)),
          pl.BlockSpec((BLKN, ca), lambda i: (i, 0)),
          pl.BlockSpec(wxt.shape, lambda i: (0, 0)),
          pl.BlockSpec((1, 3 * ga), lambda i: (0, 0)),
          pl.BlockSpec(wet.shape, lambda i: (0, 0)),
          pl.BlockSpec((1, 3 * gb), lambda i: (0, 0)),
          pl.BlockSpec(wyt.shape, lambda i: (0, 0)),
      ],
      out_specs=[pl.BlockSpec((BLKN, cn), lambda i: (i, 0)),
                 pl.BlockSpec((BLKN, oy), lambda i: (i, 0))],
      out_shape=[jax.ShapeDtypeStruct((n, cn), _f32),
                 jax.ShapeDtypeStruct((n, oy), _f32)],
  )(xx, a0, a1, wxt, bx, wet, be, wyt)


def _tc_node5(xx, a0, a1, wxt, bx, wet, be):
  n, ci = xx.shape
  ca = a0.shape[1]
  ox = wxt.shape[1]
  oe = wet.shape[1]
  cn = ox + oe

  def body(x_ref, a0_ref, a1_ref, wx_ref, bx_ref, we_ref, be_ref,
           ox_ref, os_ref):
    i = pl.program_id(0)
    xv = x_ref[...]
    agg = a0_ref[...] + a1_ref[...]
    xa = jnp.maximum(
        jnp.dot(xv, wx_ref[...], preferred_element_type=_f32) + bx_ref[...],
        0.0)
    xe = jnp.maximum(
        jnp.dot(agg, we_ref[...], preferred_element_type=_f32) + be_ref[...],
        0.0)
    xn = jnp.concatenate([xa, xe], axis=1)
    ox_ref[...] = xn

    @pl.when(i == 0)
    def _():
      os_ref[...] = jnp.zeros_like(os_ref)

    os_ref[...] += jnp.sum(xn, axis=0, keepdims=True)

  return pl.pallas_call(
      body,
      grid=(n // BLKN,),
      in_specs=[
          pl.BlockSpec((BLKN, ci), lambda i: (i, 0)),
          pl.BlockSpec((BLKN, ca), lambda i: (i, 0)),
          pl.BlockSpec((BLKN, ca), lambda i: (i, 0)),
          pl.BlockSpec(wxt.shape, lambda i: (0, 0)),
          pl.BlockSpec((1, ox), lambda i: (0, 0)),
          pl.BlockSpec(wet.shape, lambda i: (0, 0)),
          pl.BlockSpec((1, oe), lambda i: (0, 0)),
      ],
      out_specs=[pl.BlockSpec((BLKN, cn), lambda i: (i, 0)),
                 pl.BlockSpec((1, cn), lambda i: (0, 0))],
      out_shape=[jax.ShapeDtypeStruct((n, cn), _f32),
                 jax.ShapeDtypeStruct((1, cn), _f32)],
  )(xx, a0, a1, wxt, bx, wet, be)


def _tc_mlp(xx, st, w1t, b1, w2t, b2, w3t, b3):
  n, c = xx.shape

  def body(x_ref, st_ref, w1_ref, b1_ref, w2_ref, b2_ref, w3_ref, b3_ref,
           o_ref):
    s = jnp.broadcast_to(st_ref[...], (BLKN, c))
    q = jnp.concatenate([s, x_ref[...]], axis=1)
    q = jnp.maximum(jnp.dot(q, w1_ref[...], preferred_element_type=_f32)
                    + b1_ref[...], 0.0)
    q = jnp.maximum(jnp.dot(q, w2_ref[...], preferred_element_type=_f32)
                    + b2_ref[...], 0.0)
    q = jnp.maximum(jnp.dot(q, w3_ref[...], preferred_element_type=_f32)
                    + b3_ref[...], 0.0)
    o_ref[...] = q

  return pl.pallas_call(
      body,
      grid=(n // BLKN,),
      in_specs=[
          pl.BlockSpec((BLKN, c), lambda i: (i, 0)),
          pl.BlockSpec((1, c), lambda i: (0, 0)),
          pl.BlockSpec(w1t.shape, lambda i: (0, 0)),
          pl.BlockSpec((1, w1t.shape[1]), lambda i: (0, 0)),
          pl.BlockSpec(w2t.shape, lambda i: (0, 0)),
          pl.BlockSpec((1, w2t.shape[1]), lambda i: (0, 0)),
          pl.BlockSpec(w3t.shape, lambda i: (0, 0)),
          pl.BlockSpec((1, w3t.shape[1]), lambda i: (0, 0)),
      ],
      out_specs=pl.BlockSpec((BLKN, 1), lambda i: (i, 0)),
      out_shape=jax.ShapeDtypeStruct((n, 1), _f32),
  )(xx, st, w1t, b1, w2t, b2, w3t, b3)


_EDGE_CFG = {
    'e1': (10, 1, 2, True, True),
    'e2': (18, 4, 12, True, True),
    'e3': (18, 10, 12, True, True),
    'e4': (18, 10, 12, True, True),
    'e5': (3, 10, 3, False, False),
}

_EDGE_KERNELS = {tag: _edge_sc(*cfg) for tag, cfg in _EDGE_CFG.items()}


def _pool_perm(o):
  g = o // 3
  return jnp.asarray([3 * (j % g) + j // g for j in range(o)], _i32)


def _pack_w(p, tag):
  cy, ce, oe, _, _ = _EDGE_CFG[tag]
  pw = oe * ce + oe + cy
  pp = ((pw + 15) // 16) * 16
  vec = jnp.concatenate([p[tag + 'e_w'].reshape(-1), p[tag + 'e_b'],
                         p[tag + 'x_b']])
  return jnp.concatenate([vec, jnp.zeros((pp - pw,), _f32)])


def _pad_y(y):
  return jnp.concatenate([y, jnp.zeros((NP - N_NODES, y.shape[1]), _f32)],
                         axis=0)


def kernel(x, edge_index, edge_attr, params):
  p = params
  x = x.astype(_f32)
  ei = edge_index.astype(_i32)
  e = ei.shape[1]
  pad = EP - e
  i0 = jnp.concatenate([ei[0], jnp.full((pad,), N_NODES, _i32)])
  i1 = jnp.concatenate([ei[1], jnp.full((pad,), N_NODES, _i32)])
  i0 = i0.reshape(EP // CHUNK, CHUNK)
  i1 = i1.reshape(EP // CHUNK, CHUNK)
  ea = jnp.concatenate([edge_attr.astype(_f32),
                        jnp.zeros((pad, edge_attr.shape[1]), _f32)], axis=0)

  def edge(tag, y, ea_in):
    cy, ce, oe, do_pool, write_ea = _EDGE_CFG[tag]
    cout = (cy + oe) // 3 if do_pool else (cy + oe)
    z = jnp.zeros((NP, cout), _f32)
    outs = _EDGE_KERNELS[tag](_pad_y(y), i0, i1, ea_in.reshape(-1),
                              _pack_w(p, tag), z)
    if not isinstance(outs, (list, tuple)):
      outs = (outs,)
    if write_ea:
      return outs[0], outs[1]
    return None, outs[0]

  def node(tag, xx, aggp, ynext_w):
    pa = _pool_perm(p[tag + 'x_w'].shape[0])
    pb = _pool_perm(p[tag + 'e_w'].shape[0])
    return _tc_node(
        xx, aggp[0, :N_NODES], aggp[1, :N_NODES],
        p[tag + 'x_w'][pa].T, p[tag + 'x_b'][pa].reshape(1, -1),
        p[tag + 'e_w'][pb].T, p[tag + 'e_b'][pb].reshape(1, -1),
        ynext_w.T)

  y = _tc_linear(x, p['e1x_w'].T)
  ea1, agg = edge('e1', y, ea)
  x1, y = node('n1', x, agg, p['e2x_w'])
  ea2, agg = edge('e2', y, ea1)
  x2, y = node('n2', x1, agg, p['e3x_w'])
  ea3, agg = edge('e3', y, ea2)
  x3, y = node('n3', x2, agg, p['e4x_w'])
  ea4, agg = edge('e4', y, ea3)
  x4, y = node('n4', x3, agg, p['e5x_w'])
  _, agg = edge('e5', y, ea4)
  x5, st = _tc_node5(
      x4, agg[0, :N_NODES], agg[1, :N_NODES],
      p['n5x_w'].T, p['n5x_b'].reshape(1, -1),
      p['n5e_w'].T, p['n5e_b'].reshape(1, -1))
  q = _tc_mlp(x5, st,
              p['fc1_w'].T, p['fc1_b'].reshape(1, -1),
              p['fc2_w'].T, p['fc2_b'].reshape(1, -1),
              p['fc3_w'].T, p['fc3_b'].reshape(1, -1))
  return q.reshape(-1)


# speedup vs baseline: 5.4789x; 1.0265x over previous
import jax
import jax.numpy as jnp
from jax import lax
from jax.experimental import pallas as pl
from jax.experimental.pallas import tpu as pltpu
from jax.experimental.pallas import tpu_sc as plsc

N_NODES = 50000
NPP = 50176
NPT32 = NPP // 32
NB = 112
CHUNK = 128
BCH = 8
BLOCK = CHUNK * BCH
NTILES = 32
CPT = 392
NBLK = CPT // BCH
EP = NTILES * CPT * CHUNK
BLKN = 2000

_f32 = jnp.float32
_i32 = jnp.int32

_CP = pltpu.CompilerParams(needs_layout_passes=False,
                           use_tc_tiling_on_sc=False)

_CFG = {
    'e1': (5, 0, 10, 1, 2, True, True, False, False),
    'e2': (5, 4, 18, 4, 12, True, True, True, True),
    'e3': (10, 10, 18, 10, 12, True, True, True, True),
    'e4': (10, 10, 18, 10, 12, True, True, True, True),
    'e5': (10, 10, 3, 10, 3, False, False, True, True),
}


def _woffs(tag):
  _, _, cy, ce, oe, _, _, _, _ = _CFG[tag]
  o = {}
  off = 0
  o['we'] = off; off += oe * ce
  o['be'] = off; off += oe
  o['bx'] = off; off += cy
  o['total'] = ((off + 15) // 16) * 16
  return o


def _wboffs(tag):
  cin, ca, cy, _, _, _, _, has_node, _ = _CFG[tag]
  cxn = 10 if has_node else cin
  o = {}
  off = 0
  o['wy'] = off; off += cy * cxn
  if has_node:
    o['wnx'] = off; off += 18 * cin
    o['bnx'] = off; off += 18
    o['wne'] = off; off += 12 * ca
    o['bne'] = off; off += 12
  o['count'] = off
  return o


def _node_sc(tag):
  cin, ca, cy, _, _, _, _, has_node, xout = _CFG[tag]
  cxn = 10 if has_node else cin
  WB = _wboffs(tag)

  mesh = plsc.VectorSubcoreMesh(core_axis_name="c", subcore_axis_name="s")
  out_type = [jax.ShapeDtypeStruct((NPP, cy), _f32)]
  if xout:
    out_type.append(jax.ShapeDtypeStruct((NPP, cxn), _f32))
  scratch = [
      pltpu.VMEM((WB['count'] * 16,), _f32),
      pltpu.VMEM((NB, cin), _f32),
      pltpu.VMEM((NB, max(ca, 1)), _f32),
      pltpu.VMEM((NB, max(ca, 1)), _f32),
      pltpu.VMEM((NB, cy), _f32),
      pltpu.VMEM((NB, cxn), _f32),
  ]

  def body(*refs):
    it = iter(refs)
    xp_h = next(it)
    aggin_h = next(it) if has_node else None
    wb_h = next(it)
    y_h = next(it)
    xout_h = next(it) if xout else None
    wbv, xbuf, a0b, a1b, ybuf, xnb = list(it)

    cid = lax.axis_index("c")
    sid = lax.axis_index("s")
    wid = sid * 2 + cid
    pltpu.sync_copy(wb_h, wbv)

    def wv(i):
      return wbv[pl.ds(i * 16, 16)]

    iota16 = lax.iota(_i32, 16)
    nbase = wid * NPT32

    def node_chunk(nc, c0):
      s = nbase + nc * NB
      pltpu.sync_copy(xp_h.at[pl.ds(s, NB)], xbuf)
      if has_node:
        pltpu.sync_copy(aggin_h.at[0, pl.ds(s, NB)], a0b)
        pltpu.sync_copy(aggin_h.at[1, pl.ds(s, NB)], a1b)

      def ngroup(g, c1):
        rows = g * 16 + iota16
        xs = [plsc.load_gather(xbuf, [rows, jnp.full((16,), c, _i32)])
              for c in range(cin)]
        if has_node:
          ags = [plsc.load_gather(a0b, [rows, jnp.full((16,), c, _i32)])
                 + plsc.load_gather(a1b, [rows, jnp.full((16,), c, _i32)])
                 for c in range(ca)]

          def nch(ch):
            if ch < 18:
              v = wv(WB['bnx'] + ch)
              for c in range(cin):
                v = v + xs[c] * wv(WB['wnx'] + ch * cin + c)
            else:
              j = ch - 18
              v = wv(WB['bne'] + j)
              for c in range(ca):
                v = v + ags[c] * wv(WB['wne'] + j * ca + c)
            return v

          xn = []
          for k in range(cxn):
            v = jnp.maximum(jnp.maximum(nch(3 * k), nch(3 * k + 1)),
                            nch(3 * k + 2))
            v = jnp.maximum(v, 0.0)
            xn.append(v)
            plsc.store_scatter(xnb, [rows, jnp.full((16,), k, _i32)], v)
        else:
          xn = xs
        for j in range(cy):
          yv = xn[0] * wv(WB['wy'] + j * cxn)
          for c in range(1, cxn):
            yv = yv + xn[c] * wv(WB['wy'] + j * cxn + c)
          plsc.store_scatter(ybuf, [rows, jnp.full((16,), j, _i32)], yv)
        return c1

      lax.fori_loop(0, NB // 16, ngroup, 0)
      pltpu.sync_copy(ybuf, y_h.at[pl.ds(s, NB)])
      if xout:
        pltpu.sync_copy(xnb, xout_h.at[pl.ds(s, NB)])
      return c0

    lax.fori_loop(0, NPT32 // NB, node_chunk, 0)

  return pl.kernel(body, out_type=out_type, mesh=mesh,
                   scratch_types=scratch, compiler_params=_CP)


def _edge_sc(tag):
  _, _, cy, ce, oe, pool, write_ea, _, _ = _CFG[tag]
  cpre = cy + oe
  cout = cpre // 3 if pool else cpre
  W = _woffs(tag)
  Pp = W['total']
  mesh = plsc.VectorSubcoreMesh(core_axis_name="c", subcore_axis_name="s")

  out_type = []
  if write_ea:
    out_type.append(jax.ShapeDtypeStruct((EP, cout), _f32))
  out_type.append(jax.ShapeDtypeStruct((2, NPP, cout), _f32))

  scratch = [
      pltpu.VMEM((BCH, CHUNK), _i32),
      pltpu.VMEM((BCH, CHUNK), _i32),
      pltpu.VMEM((BLOCK * ce,), _f32),
      pltpu.VMEM((BLOCK, cy), _f32),
      pltpu.VMEM((BLOCK, cy), _f32),
      pltpu.VMEM((BLOCK, cout), _f32),
      pltpu.VMEM((Pp,), _f32),
      pltpu.VMEM_SHARED((NPP, cout), _f32),
      pltpu.SemaphoreType.DMA,
  ]

  def body(*refs):
    it = iter(refs)
    y_h = next(it)
    i0_h = next(it)
    i1_h = next(it)
    ea_h = next(it)
    w_h = next(it)
    z_h = next(it)
    eaout_h = next(it) if write_ea else None
    agg_h = next(it)
    idx0, idx1, eab, r0, r1, outb, wvm, aggs, sem = list(it)

    cid = lax.axis_index("c")
    sid = lax.axis_index("s")
    wid = sid * 2 + cid

    pltpu.sync_copy(w_h, wvm)
    ws = []
    for k in range(Pp // 16):
      v = wvm[pl.ds(k * 16, 16)]
      ws.extend(v[j] for j in range(16))

    @pl.when(sid == 0)
    def _zero():
      pltpu.sync_copy(z_h, aggs)

    plsc.subcore_barrier()

    iota16 = lax.iota(_i32, 16)
    tile_ch = wid * CPT

    def block_body(blk, c0):
      ch0 = tile_ch + blk * BCH
      e0 = ch0 * CHUNK
      pltpu.sync_copy(i0_h.at[pl.ds(ch0, BCH)], idx0)
      pltpu.sync_copy(i1_h.at[pl.ds(ch0, BCH)], idx1)
      pltpu.sync_copy(ea_h.at[pl.ds(e0 * ce, BLOCK * ce)], eab)
      descs = []
      for k in range(BCH):
        descs.append(pltpu.async_copy(
            y_h.at[idx0.at[k]], r0.at[pl.ds(k * CHUNK, CHUNK)], sem))
        descs.append(pltpu.async_copy(
            y_h.at[idx1.at[k]], r1.at[pl.ds(k * CHUNK, CHUNK)], sem))
      for d in descs:
        d.wait()

      def group(g, c1):
        rows = g * 16 + iota16
        erows = rows * ce
        hs = [jnp.zeros((16,), _f32) + ws[W['be'] + j] for j in range(oe)]
        for c in range(ce):
          a = plsc.load_gather(eab, [erows + c])
          for j in range(oe):
            hs[j] = hs[j] + a * ws[W['we'] + j * ce + c]

        def uch(i):
          if i >= cy:
            return hs[i - cy]
          a0 = plsc.load_gather(r0, [rows, jnp.full((16,), i, _i32)])
          a1 = plsc.load_gather(r1, [rows, jnp.full((16,), i, _i32)])
          return a0 + a1 + ws[W['bx'] + i]

        for k in range(cout):
          if pool:
            v = jnp.maximum(jnp.maximum(uch(3 * k), uch(3 * k + 1)),
                            uch(3 * k + 2))
          else:
            v = uch(k)
          v = jnp.maximum(v, 0.0)
          plsc.store_scatter(outb, [rows, jnp.full((16,), k, _i32)], v)
        return c1

      lax.fori_loop(0, BLOCK // 16, group, 0)
      if write_ea:
        pltpu.sync_copy(outb, eaout_h.at[pl.ds(e0, BLOCK)])
      for k in range(BCH):
        pltpu.sync_copy(outb.at[pl.ds(k * CHUNK, CHUNK)],
                        aggs.at[idx0.at[k]], add=True)
      return c0

    lax.fori_loop(0, NBLK, block_body, 0)
    plsc.subcore_barrier()

    @pl.when(sid == 0)
    def _flush():
      pltpu.sync_copy(aggs, agg_h.at[cid])

  return pl.kernel(body, out_type=out_type, mesh=mesh,
                   scratch_types=scratch, compiler_params=_CP)


_NODE_KERNELS = {tag: _node_sc(tag) for tag in _CFG}
_EDGE_KERNELS = {tag: _edge_sc(tag) for tag in _CFG}


def _tc_node5(xx, a0, a1, wxt, bx, wet, be):
  n, ci = xx.shape
  ca = a0.shape[1]
  ox = wxt.shape[1]
  oe = wet.shape[1]
  cn = ox + oe

  def body(x_ref, a0_ref, a1_ref, wx_ref, bx_ref, we_ref, be_ref,
           ox_ref, os_ref):
    i = pl.program_id(0)
    xv = x_ref[...]
    agg = a0_ref[...] + a1_ref[...]
    xa = jnp.maximum(
        jnp.dot(xv, wx_ref[...], preferred_element_type=_f32) + bx_ref[...],
        0.0)
    xe = jnp.maximum(
        jnp.dot(agg, we_ref[...], preferred_element_type=_f32) + be_ref[...],
        0.0)
    xn = jnp.concatenate([xa, xe], axis=1)
    ox_ref[...] = xn

    @pl.when(i == 0)
    def _():
      os_ref[...] = jnp.zeros_like(os_ref)

    os_ref[...] += jnp.sum(xn, axis=0, keepdims=True)

  return pl.pallas_call(
      body,
      grid=(n // BLKN,),
      in_specs=[
          pl.BlockSpec((BLKN, ci), lambda i: (i, 0)),
          pl.BlockSpec((BLKN, ca), lambda i: (i, 0)),
          pl.BlockSpec((BLKN, ca), lambda i: (i, 0)),
          pl.BlockSpec(wxt.shape, lambda i: (0, 0)),
          pl.BlockSpec((1, ox), lambda i: (0, 0)),
          pl.BlockSpec(wet.shape, lambda i: (0, 0)),
          pl.BlockSpec((1, oe), lambda i: (0, 0)),
      ],
      out_specs=[pl.BlockSpec((BLKN, cn), lambda i: (i, 0)),
                 pl.BlockSpec((1, cn), lambda i: (0, 0))],
      out_shape=[jax.ShapeDtypeStruct((n, cn), _f32),
                 jax.ShapeDtypeStruct((1, cn), _f32)],
  )(xx, a0, a1, wxt, bx, wet, be)


def _tc_mlp(xx, st, w1t, b1, w2t, b2, w3t, b3):
  n, c = xx.shape

  def body(x_ref, st_ref, w1_ref, b1_ref, w2_ref, b2_ref, w3_ref, b3_ref,
           o_ref):
    s = jnp.broadcast_to(st_ref[...], (BLKN, c))
    q = jnp.concatenate([s, x_ref[...]], axis=1)
    q = jnp.maximum(jnp.dot(q, w1_ref[...], preferred_element_type=_f32)
                    + b1_ref[...], 0.0)
    q = jnp.maximum(jnp.dot(q, w2_ref[...], preferred_element_type=_f32)
                    + b2_ref[...], 0.0)
    q = jnp.maximum(jnp.dot(q, w3_ref[...], preferred_element_type=_f32)
                    + b3_ref[...], 0.0)
    o_ref[...] = q

  return pl.pallas_call(
      body,
      grid=(n // BLKN,),
      in_specs=[
          pl.BlockSpec((BLKN, c), lambda i: (i, 0)),
          pl.BlockSpec((1, c), lambda i: (0, 0)),
          pl.BlockSpec(w1t.shape, lambda i: (0, 0)),
          pl.BlockSpec((1, w1t.shape[1]), lambda i: (0, 0)),
          pl.BlockSpec(w2t.shape, lambda i: (0, 0)),
          pl.BlockSpec((1, w2t.shape[1]), lambda i: (0, 0)),
          pl.BlockSpec(w3t.shape, lambda i: (0, 0)),
          pl.BlockSpec((1, w3t.shape[1]), lambda i: (0, 0)),
      ],
      out_specs=pl.BlockSpec((BLKN, 1), lambda i: (i, 0)),
      out_shape=jax.ShapeDtypeStruct((n, 1), _f32),
  )(xx, st, w1t, b1, w2t, b2, w3t, b3)


def _pack_w(p, tag):
  W = _woffs(tag)
  vec = jnp.concatenate([p[tag + 'e_w'].reshape(-1), p[tag + 'e_b'],
                         p[tag + 'x_b']])
  return jnp.concatenate([vec, jnp.zeros((W['total'] - vec.shape[0],), _f32)])


def _pack_wb(p, tag, node_tag):
  _, _, _, _, _, _, _, has_node, _ = _CFG[tag]
  parts = [p[tag + 'x_w'].reshape(-1)]
  if has_node:
    parts += [p[node_tag + 'x_w'].reshape(-1), p[node_tag + 'x_b'],
              p[node_tag + 'e_w'].reshape(-1), p[node_tag + 'e_b']]
  return jnp.repeat(jnp.concatenate(parts), 16)


def kernel(x, edge_index, edge_attr, params):
  p = params
  x = x.astype(_f32)
  ei = edge_index.astype(_i32)
  e = ei.shape[1]
  pad = EP - e
  i0 = jnp.concatenate([ei[0], jnp.full((pad,), N_NODES, _i32)])
  i1 = jnp.concatenate([ei[1], jnp.full((pad,), N_NODES, _i32)])
  i0 = i0.reshape(EP // CHUNK, CHUNK)
  i1 = i1.reshape(EP // CHUNK, CHUNK)
  ea = jnp.concatenate([edge_attr.astype(_f32).reshape(-1),
                        jnp.zeros((pad * edge_attr.shape[1],), _f32)])
  xp0 = jnp.concatenate([x, jnp.zeros((NPP - N_NODES, x.shape[1]), _f32)],
                        axis=0)

  def run(tag, node_tag, xp, aggp, ea_in):
    _, _, cy, _, oe, pool, write_ea, has_node, xout = _CFG[tag]
    cout = (cy + oe) // 3 if pool else (cy + oe)
    if has_node:
      aggsum = aggp[0] + aggp[1]
      xa = jnp.maximum(xp @ p[node_tag + 'x_w'].T + p[node_tag + 'x_b'], 0.0)
      xe = jnp.maximum(aggsum @ p[node_tag + 'e_w'].T + p[node_tag + 'e_b'], 0.0)
      u = jnp.concatenate([xa, xe], axis=1)
      xn = u.reshape(NPP, 10, 3).max(axis=2)
    else:
      xn = xp
    y = xn @ p[tag + 'x_w'].T
    xo = xn if xout else None

    z = jnp.zeros((NPP, cout), _f32)
    eargs = [y, i0, i1, ea_in.reshape(-1), _pack_w(p, tag), z]
    outs = _EDGE_KERNELS[tag](*eargs)
    if not isinstance(outs, (list, tuple)):
      outs = (outs,)
    outs = list(outs)
    eao = outs.pop(0) if write_ea else None
    return xo, eao, outs[0]

  _, ea1, agg = run('e1', None, xp0, None, ea)
  x1p, ea2, agg = run('e2', 'n1', xp0, agg, ea1)
  x2p, ea3, agg = run('e3', 'n2', x1p, agg, ea2)
  x3p, ea4, agg = run('e4', 'n3', x2p, agg, ea3)
  x4p, _, agg = run('e5', 'n4', x3p, agg, ea4)

  x5, st = _tc_node5(
      x4p[:N_NODES], agg[0, :N_NODES], agg[1, :N_NODES],
      p['n5x_w'].T, p['n5x_b'].reshape(1, -1),
      p['n5e_w'].T, p['n5e_b'].reshape(1, -1))
  q = _tc_mlp(x5, st,
              p['fc1_w'].T, p['fc1_b'].reshape(1, -1),
              p['fc2_w'].T, p['fc2_b'].reshape(1, -1),
              p['fc3_w'].T, p['fc3_b'].reshape(1, -1))
  return q.reshape(-1)
